# baseline (speedup 1.0000x reference)
import functools

import jax
import jax.numpy as jnp
from jax.experimental import pallas as pl
from jax.experimental.pallas import tpu as pltpu

_K = 1024
_D = 64
_B = 64
_HW = 576
_N = _B * _HW
_COMMIT = 0.25


def _vq_block(x_ref, e_ref, et_ref, en_ref, q_ref, loss_ref, ppl_ref,
              cnt_ref, sse_ref):
    b = pl.program_id(0)

    @pl.when(b == 0)
    def _init():
        cnt_ref[...] = jnp.zeros_like(cnt_ref)
        sse_ref[...] = jnp.zeros_like(sse_ref)

    x = x_ref[0]
    e = e_ref[...]
    fnorm = jnp.sum(x * x, axis=0)
    c = jax.lax.dot_general(e, x, (((1,), (0,)), ((), ())),
                            preferred_element_type=jnp.float32)
    dist = (en_ref[...] + fnorm[None, :]) - 2.0 * c
    minv = jnp.min(dist, axis=0)
    kiota = jax.lax.broadcasted_iota(jnp.int32, (_K, _HW), 0).astype(jnp.float32)
    sel = jnp.where(dist == minv[None, :], kiota, jnp.float32(_K))
    idx = jnp.min(sel, axis=0)
    one_hot = (sel == idx[None, :]).astype(jnp.bfloat16)
    q = jax.lax.dot_general(et_ref[...], one_hot, (((1,), (0,)), ((), ())),
                            preferred_element_type=jnp.float32)
    diff = q - x
    q_ref[0] = x + diff
    sse_ref[...] = sse_ref[...] + jnp.sum(diff * diff)
    ones = jnp.ones((_HW, 8), jnp.bfloat16)
    cnt_ref[...] = cnt_ref[...] + jax.lax.dot_general(
        one_hot, ones, (((1,), (0,)), ((), ())),
        preferred_element_type=jnp.float32)

    @pl.when(b == _B - 1)
    def _finalize():
        m = sse_ref[...] / jnp.float32(_N * _D)
        loss_ref[...] = m + _COMMIT * m
        p = cnt_ref[:, 0] / jnp.float32(_N)
        ent = jnp.sum(p * jnp.log(p + 1e-10))
        ppl_ref[...] = jnp.exp(-ent) * jnp.ones_like(ppl_ref)


@functools.partial(jax.jit)
def kernel(inputs, embedding_weight):
    x2 = inputs.reshape(_B, _D, _HW)
    et = embedding_weight.T.astype(jnp.bfloat16)
    q2, loss, ppl = pl.pallas_call(
        _vq_block,
        grid=(_B,),
        in_specs=[
            pl.BlockSpec((1, _D, _HW), lambda b: (b, 0, 0)),
            pl.BlockSpec((_K, _D), lambda b: (0, 0)),
            pl.BlockSpec((_D, _K), lambda b: (0, 0)),
            pl.BlockSpec((_K, 1), lambda b: (0, 0)),
        ],
        out_specs=[
            pl.BlockSpec((1, _D, _HW), lambda b: (b, 0, 0)),
            pl.BlockSpec((1, 1), lambda b: (0, 0)),
            pl.BlockSpec((1, 1), lambda b: (0, 0)),
        ],
        out_shape=[
            jax.ShapeDtypeStruct((_B, _D, _HW), jnp.float32),
            jax.ShapeDtypeStruct((1, 1), jnp.float32),
            jax.ShapeDtypeStruct((1, 1), jnp.float32),
        ],
        scratch_shapes=[
            pltpu.VMEM((_K, 8), jnp.float32),
            pltpu.VMEM((1, 1), jnp.float32),
        ],
    )(x2, embedding_weight, et,
      jnp.sum(embedding_weight**2, axis=1, keepdims=True))
    return loss[0, 0], q2.reshape(_B, _D, 24, 24), ppl[0, 0]


# speedup vs baseline: 1.4403x; 1.4403x over previous
import functools

import jax
import jax.numpy as jnp
from jax.experimental import pallas as pl
from jax.experimental.pallas import tpu as pltpu

_K = 1024
_D = 64
_B = 64
_HW = 576
_BB = 4
_W = _BB * _HW
_G = _B // _BB
_N = _B * _HW
_COMMIT = 0.25


def _vq_block(x_ref, e_ref, et_ref, en_ref, q_ref, loss_ref, ppl_ref,
              oh_ref, sse_ref):
    b = pl.program_id(0)

    @pl.when(b == 0)
    def _init():
        oh_ref[...] = jnp.zeros_like(oh_ref)
        sse_ref[...] = jnp.zeros_like(sse_ref)

    x = jnp.concatenate([x_ref[i] for i in range(_BB)], axis=1)
    e = e_ref[...]
    fnorm = jnp.sum(x * x, axis=0)
    c = jax.lax.dot_general(e, x, (((1,), (0,)), ((), ())),
                            preferred_element_type=jnp.float32)
    dist = (en_ref[...] + fnorm[None, :]) - 2.0 * c
    minv = jnp.min(dist, axis=0)
    kiota = jax.lax.broadcasted_iota(jnp.int32, (_K, _W), 0).astype(jnp.float32)
    sel = jnp.where(dist == minv[None, :], kiota, jnp.float32(_K))
    idx = jnp.min(sel, axis=0)
    one_hot = (sel == idx[None, :]).astype(jnp.bfloat16)
    q = jax.lax.dot_general(et_ref[...], one_hot, (((1,), (0,)), ((), ())),
                            preferred_element_type=jnp.float32)
    diff = q - x
    qst = x + diff
    for i in range(_BB):
        q_ref[i] = qst[:, i * _HW:(i + 1) * _HW]
    sse_ref[...] = sse_ref[...] + jnp.sum(diff * diff)
    oh_ref[...] = oh_ref[...] + one_hot

    @pl.when(b == _G - 1)
    def _finalize():
        m = sse_ref[...] / jnp.float32(_N * _D)
        loss_ref[...] = m + _COMMIT * m
        ones = jnp.ones((_W, 8), jnp.bfloat16)
        cnt = jax.lax.dot_general(
            oh_ref[...], ones, (((1,), (0,)), ((), ())),
            preferred_element_type=jnp.float32)
        p = cnt[:, 0] / jnp.float32(_N)
        ent = jnp.sum(p * jnp.log(p + 1e-10))
        ppl_ref[...] = jnp.exp(-ent) * jnp.ones_like(ppl_ref)


@functools.partial(jax.jit)
def kernel(inputs, embedding_weight):
    x2 = inputs.reshape(_B, _D, _HW)
    et = embedding_weight.T.astype(jnp.bfloat16)
    q2, loss, ppl = pl.pallas_call(
        _vq_block,
        grid=(_G,),
        in_specs=[
            pl.BlockSpec((_BB, _D, _HW), lambda b: (b, 0, 0)),
            pl.BlockSpec((_K, _D), lambda b: (0, 0)),
            pl.BlockSpec((_D, _K), lambda b: (0, 0)),
            pl.BlockSpec((_K, 1), lambda b: (0, 0)),
        ],
        out_specs=[
            pl.BlockSpec((_BB, _D, _HW), lambda b: (b, 0, 0)),
            pl.BlockSpec((1, 1), lambda b: (0, 0)),
            pl.BlockSpec((1, 1), lambda b: (0, 0)),
        ],
        out_shape=[
            jax.ShapeDtypeStruct((_B, _D, _HW), jnp.float32),
            jax.ShapeDtypeStruct((1, 1), jnp.float32),
            jax.ShapeDtypeStruct((1, 1), jnp.float32),
        ],
        scratch_shapes=[
            pltpu.VMEM((_K, _W), jnp.bfloat16),
            pltpu.VMEM((1, 1), jnp.float32),
        ],
    )(x2, embedding_weight, et,
      jnp.sum(embedding_weight**2, axis=1, keepdims=True))
    return loss[0, 0], q2.reshape(_B, _D, 24, 24), ppl[0, 0]


# speedup vs baseline: 1.4406x; 1.0002x over previous
import functools

import jax
import jax.numpy as jnp
from jax.experimental import pallas as pl
from jax.experimental.pallas import tpu as pltpu

_K = 1024
_D = 64
_B = 64
_HW = 576
_BB = 8
_W = _BB * _HW
_G = _B // _BB
_N = _B * _HW
_COMMIT = 0.25


def _vq_block(x_ref, e_ref, et_ref, en_ref, q_ref, loss_ref, ppl_ref,
              oh_ref, sse_ref):
    b = pl.program_id(0)

    @pl.when(b == 0)
    def _init():
        oh_ref[...] = jnp.zeros_like(oh_ref)
        sse_ref[...] = jnp.zeros_like(sse_ref)

    x = jnp.concatenate([x_ref[i] for i in range(_BB)], axis=1)
    e = e_ref[...]
    fnorm = jnp.sum(x * x, axis=0)
    c = jax.lax.dot_general(e, x, (((1,), (0,)), ((), ())),
                            preferred_element_type=jnp.float32)
    dist = (en_ref[...] + fnorm[None, :]) - 2.0 * c
    minv = jnp.min(dist, axis=0)
    kiota = jax.lax.broadcasted_iota(jnp.int32, (_K, _W), 0).astype(jnp.float32)
    sel = jnp.where(dist == minv[None, :], kiota, jnp.float32(_K))
    idx = jnp.min(sel, axis=0)
    one_hot = (sel == idx[None, :]).astype(jnp.bfloat16)
    q = jax.lax.dot_general(et_ref[...], one_hot, (((1,), (0,)), ((), ())),
                            preferred_element_type=jnp.float32)
    diff = q - x
    qst = x + diff
    for i in range(_BB):
        q_ref[i] = qst[:, i * _HW:(i + 1) * _HW]
    sse_ref[...] = sse_ref[...] + jnp.sum(diff * diff)
    oh_ref[...] = oh_ref[...] + one_hot

    @pl.when(b == _G - 1)
    def _finalize():
        m = sse_ref[...] / jnp.float32(_N * _D)
        loss_ref[...] = m + _COMMIT * m
        ones = jnp.ones((_W, 8), jnp.bfloat16)
        cnt = jax.lax.dot_general(
            oh_ref[...], ones, (((1,), (0,)), ((), ())),
            preferred_element_type=jnp.float32)
        p = cnt[:, 0] / jnp.float32(_N)
        ent = jnp.sum(p * jnp.log(p + 1e-10))
        ppl_ref[...] = jnp.exp(-ent) * jnp.ones_like(ppl_ref)


@functools.partial(jax.jit)
def kernel(inputs, embedding_weight):
    x2 = inputs.reshape(_B, _D, _HW)
    et = embedding_weight.T.astype(jnp.bfloat16)
    q2, loss, ppl = pl.pallas_call(
        _vq_block,
        grid=(_G,),
        in_specs=[
            pl.BlockSpec((_BB, _D, _HW), lambda b: (b, 0, 0)),
            pl.BlockSpec((_K, _D), lambda b: (0, 0)),
            pl.BlockSpec((_D, _K), lambda b: (0, 0)),
            pl.BlockSpec((_K, 1), lambda b: (0, 0)),
        ],
        out_specs=[
            pl.BlockSpec((_BB, _D, _HW), lambda b: (b, 0, 0)),
            pl.BlockSpec((1, 1), lambda b: (0, 0)),
            pl.BlockSpec((1, 1), lambda b: (0, 0)),
        ],
        out_shape=[
            jax.ShapeDtypeStruct((_B, _D, _HW), jnp.float32),
            jax.ShapeDtypeStruct((1, 1), jnp.float32),
            jax.ShapeDtypeStruct((1, 1), jnp.float32),
        ],
        scratch_shapes=[
            pltpu.VMEM((_K, _W), jnp.bfloat16),
            pltpu.VMEM((1, 1), jnp.float32),
        ],
    )(x2, embedding_weight, et,
      jnp.sum(embedding_weight**2, axis=1, keepdims=True))
    return loss[0, 0], q2.reshape(_B, _D, 24, 24), ppl[0, 0]
